# Initial kernel scaffold; baseline (speedup 1.0000x reference)
import jax
import jax.numpy as jnp
from jax.experimental import pallas as pl


def kernel(x, table):
    raise NotImplementedError("write your pallas kernel here")


# speedup vs baseline: 2.8118x; 2.8118x over previous
import functools

import jax
import jax.numpy as jnp
from jax import lax
from jax.experimental import pallas as pl
from jax.experimental.pallas import tpu as pltpu
from jax.experimental.pallas import tpu_sc as plsc

D_MODEL = 128
CHUNK = 128


@functools.lru_cache(maxsize=None)
def _make_gather(n_idx: int, vocab: int, d: int):
    info = plsc.get_sparse_core_info()
    nc, ns = info.num_cores, info.num_subcores
    nw = nc * ns
    assert n_idx % (nw * CHUNK) == 0
    per_w = n_idx // nw
    n_chunks = per_w // CHUNK
    mesh = plsc.VectorSubcoreMesh(core_axis_name="c", subcore_axis_name="s")

    @functools.partial(
        pl.kernel,
        mesh=mesh,
        out_type=jax.ShapeDtypeStruct((n_idx, d), jnp.float32),
        scratch_types=[
            pltpu.VMEM((CHUNK,), jnp.int32),
            pltpu.VMEM((CHUNK, d), jnp.float32),
            pltpu.SemaphoreType.DMA,
        ],
    )
    def gather_kernel(table_hbm, idx_hbm, out_hbm, idx_v, rows_v, gsem):
        wid = lax.axis_index("s") * nc + lax.axis_index("c")
        base = wid * per_w

        def body(g, carry):
            off = base + g * CHUNK
            pltpu.sync_copy(idx_hbm.at[pl.ds(off, CHUNK)], idx_v)
            pltpu.async_copy(table_hbm.at[idx_v], rows_v, gsem).wait()
            pltpu.sync_copy(rows_v, out_hbm.at[pl.ds(off, CHUNK)])
            return carry

        lax.fori_loop(0, n_chunks, body, 0)

    return gather_kernel


def kernel(x, table):
    b, l = x.shape
    vocab, d = table.shape
    idx = x.reshape(-1).astype(jnp.int32)
    out = _make_gather(idx.shape[0], vocab, d)(table, idx)
    return out.reshape(b, l, d)


# speedup vs baseline: 3.4442x; 1.2249x over previous
import functools

import jax
import jax.numpy as jnp
from jax import lax
from jax.experimental import pallas as pl
from jax.experimental.pallas import tpu as pltpu
from jax.experimental.pallas import tpu_sc as plsc

CHUNK = 128
NBUF = 4


@functools.lru_cache(maxsize=None)
def _make_gather(n_idx: int, vocab: int, d: int):
    info = plsc.get_sparse_core_info()
    nc, ns = info.num_cores, info.num_subcores
    nw = nc * ns
    assert n_idx % (nw * CHUNK * NBUF) == 0
    per_w = n_idx // nw
    n_chunks = per_w // CHUNK
    n_groups = n_chunks // NBUF
    mesh = plsc.VectorSubcoreMesh(core_axis_name="c", subcore_axis_name="s")

    @functools.partial(
        pl.kernel,
        mesh=mesh,
        out_type=jax.ShapeDtypeStruct((n_idx, d), jnp.float32),
        scratch_types=[
            pltpu.VMEM((per_w,), jnp.int32),
            pltpu.VMEM((NBUF, CHUNK, d), jnp.float32),
            pltpu.SemaphoreType.DMA((NBUF,)),
            pltpu.SemaphoreType.DMA((NBUF,)),
        ],
    )
    def gather_kernel(table_hbm, idx_hbm, out_hbm, idx_v, rows_v, gsem, osem):
        wid = lax.axis_index("s") * nc + lax.axis_index("c")
        base = wid * per_w

        def gather_chunk(c, b):
            pltpu.async_copy(
                table_hbm.at[idx_v.at[pl.ds(c * CHUNK, CHUNK)]],
                rows_v.at[b], gsem.at[b])

        def gather_wait(b):
            pltpu.make_async_copy(
                table_hbm.at[idx_v.at[pl.ds(0, CHUNK)]],
                rows_v.at[b], gsem.at[b]).wait()

        def scatter_chunk(c, b):
            pltpu.async_copy(
                rows_v.at[b], out_hbm.at[pl.ds(base + c * CHUNK, CHUNK)],
                osem.at[b])

        def scatter_wait(b):
            pltpu.make_async_copy(
                rows_v.at[b], out_hbm.at[pl.ds(base, CHUNK)],
                osem.at[b]).wait()

        pltpu.sync_copy(idx_hbm.at[pl.ds(base, per_w)], idx_v)

        for b in range(NBUF):
            gather_chunk(b, b)

        def group(gi, carry):
            c0 = gi * NBUF
            for b in range(NBUF):
                gather_wait(b)
                scatter_chunk(c0 + b, b)
            for b in range(NBUF):
                nxt = c0 + b + NBUF

                @pl.when(nxt < n_chunks)
                def _():
                    scatter_wait(b)
                    gather_chunk(nxt, b)

            return carry

        lax.fori_loop(0, n_groups, group, 0)

        for b in range(NBUF):
            scatter_wait(b)

    return gather_kernel


def kernel(x, table):
    b, l = x.shape
    vocab, d = table.shape
    idx = x.reshape(-1).astype(jnp.int32)
    out = _make_gather(idx.shape[0], vocab, d)(table, idx)
    return out.reshape(b, l, d)


# speedup vs baseline: 6.2912x; 1.8266x over previous
import functools

import jax
import jax.numpy as jnp
from jax import lax
from jax.experimental import pallas as pl
from jax.experimental.pallas import tpu as pltpu
from jax.experimental.pallas import tpu_sc as plsc

CB = 2
PAD = 104
NBUF = 4


@functools.lru_cache(maxsize=None)
def _make_gather(b_total: int, l: int, vocab: int, d: int):
    info = plsc.get_sparse_core_info()
    nc, ns = info.num_cores, info.num_subcores
    nw = nc * ns
    chunk_rows = CB * l
    assert chunk_rows <= PAD and PAD % 8 == 0
    assert b_total % (nw * CB * NBUF) == 0
    per_w = b_total // nw
    n_chunks = per_w // CB
    n_groups = n_chunks // NBUF
    mesh = plsc.VectorSubcoreMesh(core_axis_name="c", subcore_axis_name="s")

    @functools.partial(
        pl.kernel,
        mesh=mesh,
        out_type=jax.ShapeDtypeStruct((b_total, l, d), jnp.float32),
        scratch_types=[
            pltpu.VMEM((n_chunks, PAD), jnp.int32),
            pltpu.VMEM((NBUF, CB * l, d), jnp.float32),
            pltpu.SemaphoreType.DMA((NBUF,)),
            pltpu.SemaphoreType.DMA((NBUF,)),
        ],
    )
    def gather_kernel(table_hbm, idx_hbm, out_hbm, idx_v, rows_v, gsem, osem):
        wid = lax.axis_index("s") * nc + lax.axis_index("c")
        base = wid * per_w

        def gather_chunk(c, b):
            pltpu.async_copy(
                table_hbm.at[idx_v.at[c, pl.ds(0, chunk_rows)]],
                rows_v.at[b], gsem.at[b])

        def gather_wait(b):
            pltpu.make_async_copy(
                table_hbm.at[idx_v.at[0, pl.ds(0, chunk_rows)]],
                rows_v.at[b], gsem.at[b]).wait()

        def scatter_chunk(c, b):
            for j in range(CB):
                pltpu.async_copy(
                    rows_v.at[b, pl.ds(j * l, l)],
                    out_hbm.at[base + c * CB + j],
                    osem.at[b])

        def scatter_wait(b):
            for j in range(CB):
                pltpu.make_async_copy(
                    rows_v.at[b, pl.ds(j * l, l)],
                    out_hbm.at[base],
                    osem.at[b]).wait()

        pltpu.sync_copy(idx_hbm.at[pl.ds(wid * n_chunks, n_chunks)], idx_v)

        for b in range(NBUF):
            gather_chunk(b, b)

        def group(gi, carry):
            c0 = gi * NBUF
            for b in range(NBUF):
                gather_wait(b)
                scatter_chunk(c0 + b, b)
            for b in range(NBUF):
                nxt = c0 + b + NBUF

                @pl.when(nxt < n_chunks)
                def _():
                    scatter_wait(b)
                    gather_chunk(nxt, b)

            return carry

        lax.fori_loop(0, n_groups, group, 0)

        for b in range(NBUF):
            scatter_wait(b)

    return gather_kernel


def kernel(x, table):
    b, l = x.shape
    vocab, d = table.shape
    chunk_rows = CB * l
    idx = x.astype(jnp.int32).reshape(-1).reshape(-1, chunk_rows)
    idx = jnp.pad(idx, ((0, 0), (0, PAD - chunk_rows)))
    return _make_gather(b, l, vocab, d)(table, idx)
